# Initial kernel scaffold; baseline (speedup 1.0000x reference)
import jax
import jax.numpy as jnp
from jax.experimental import pallas as pl


def kernel(node_sequences, edge_index, emb, W_ih_f, W_hh_f, b_ih_f, b_hh_f, W_ih_b, W_hh_b, b_ih_b, b_hh_b, W1, b1, W2, b2, Wfc, bfc):
    raise NotImplementedError("write your pallas kernel here")


# speedup vs baseline: 30.0368x; 30.0368x over previous
import functools

import jax
import jax.numpy as jnp
from jax import lax
from jax.experimental import pallas as pl
from jax.experimental.pallas import tpu as pltpu
from jax.experimental.pallas import tpu_sc as plsc

N = 50000
L = 20
E = 1600000
VOCABP = 101
H = 16

B = 1024
G = 49
NP = G * B
NC = 2
NSUB = 16
NW = NC * NSUB
KCH = 16
NGRP = 25
EC = NW * KCH * NGRP
EPAD = EC * 128
RS = NP // NSUB

def _mesh():
    return plsc.VectorSubcoreMesh(core_axis_name="c", subcore_axis_name="s",
                                  num_cores=NC, num_subcores=NSUB)


_SC_PARAMS = pltpu.CompilerParams(use_tc_tiling_on_sc=False)


def _deg_kernel(dst2d, ones8, zeros8):
    @functools.partial(
        pl.kernel,
        out_type=jax.ShapeDtypeStruct((2 * NP, 8), jnp.float32),
        mesh=_mesh(),
        compiler_params=_SC_PARAMS,
        scratch_types=[
            pltpu.VMEM_SHARED((NP, 8), jnp.float32),
            pltpu.VMEM((KCH, 128), jnp.int32),
            pltpu.VMEM((128, 8), jnp.float32),
            pltpu.SemaphoreType.DMA,
            pltpu.SemaphoreType.DMA,
        ],
    )
    def body(dst_hbm, ones_hbm, zeros_hbm, out_hbm, acc, dstb, onesb, sem_i, sem_s):
        cid = lax.axis_index("c")
        sid = lax.axis_index("s")
        wid = sid * NC + cid
        pltpu.sync_copy(ones_hbm, onesb)
        pltpu.sync_copy(zeros_hbm.at[pl.ds(sid * RS, RS), :],
                        acc.at[pl.ds(sid * RS, RS), :])
        plsc.subcore_barrier()

        def group(g, carry):
            base = wid * (KCH * NGRP) + g * KCH
            pltpu.async_copy(dst_hbm.at[pl.ds(base, KCH), :], dstb, sem_i).wait()
            hs = [pltpu.async_copy(onesb, acc.at[dstb.at[k]], sem_s, add=True)
                  for k in range(KCH)]
            for h in hs:
                h.wait()
            return carry

        lax.fori_loop(0, NGRP, group, 0)
        plsc.subcore_barrier()
        pltpu.sync_copy(acc.at[pl.ds(sid * RS, RS), :],
                        out_hbm.at[pl.ds(cid * NP + sid * RS, RS), :])

    return body(dst2d, ones8, zeros8)


def _agg_kernel(src2d, dst2d, y, zeros, D):
    @functools.partial(
        pl.kernel,
        out_type=jax.ShapeDtypeStruct((2 * NP, D), jnp.float32),
        mesh=_mesh(),
        compiler_params=_SC_PARAMS,
        scratch_types=[
            pltpu.VMEM_SHARED((NP, D), jnp.float32),
            pltpu.VMEM((KCH, 128), jnp.int32),
            pltpu.VMEM((KCH, 128), jnp.int32),
            pltpu.VMEM((KCH, 128, D), jnp.float32),
            pltpu.SemaphoreType.DMA,
            pltpu.SemaphoreType.DMA,
            pltpu.SemaphoreType.DMA,
        ],
    )
    def body(src_hbm, dst_hbm, y_hbm, zeros_hbm, out_hbm, acc, srcb, dstb, rows,
             sem_i, sem_g, sem_s):
        cid = lax.axis_index("c")
        sid = lax.axis_index("s")
        wid = sid * NC + cid
        pltpu.sync_copy(zeros_hbm.at[pl.ds(sid * RS, RS), :],
                        acc.at[pl.ds(sid * RS, RS), :])
        plsc.subcore_barrier()

        def group(g, carry):
            base = wid * (KCH * NGRP) + g * KCH
            cs = pltpu.async_copy(src_hbm.at[pl.ds(base, KCH), :], srcb, sem_i)
            cd = pltpu.async_copy(dst_hbm.at[pl.ds(base, KCH), :], dstb, sem_i)
            cs.wait()
            cd.wait()
            gs = [pltpu.async_copy(y_hbm.at[srcb.at[k]], rows.at[k], sem_g)
                  for k in range(KCH)]
            for h in gs:
                h.wait()
            ss = [pltpu.async_copy(rows.at[k], acc.at[dstb.at[k]], sem_s, add=True)
                  for k in range(KCH)]
            for h in ss:
                h.wait()
            return carry

        lax.fori_loop(0, NGRP, group, 0)
        plsc.subcore_barrier()
        pltpu.sync_copy(acc.at[pl.ds(sid * RS, RS), :],
                        out_hbm.at[pl.ds(cid * NP + sid * RS, RS), :])

    return body(src2d, dst2d, y, zeros)


def _lrelu(x):
    return jnp.where(x >= 0, x, 0.01 * x)


def _enc_body(tok_ref, emb_ref, wihf_ref, whhf_ref, bf_ref, wihb_ref, bb_ref,
              w1_ref, outact_ref, xw1_ref):
    tok = tok_ref[...]
    embf = _lrelu(emb_ref[...])
    pf = lax.dot_general(embf, wihf_ref[...], (((1,), (1,)), ((), ()))) + bf_ref[...]
    pb = lax.dot_general(embf, wihb_ref[...], (((1,), (1,)), ((), ()))) + bb_ref[...]
    iota = lax.broadcasted_iota(jnp.int32, (1, VOCABP), 1)
    whhf = whhf_ref[...]
    h = jnp.zeros((B, H), jnp.float32)
    c = jnp.zeros((B, H), jnp.float32)
    for t in range(L):
        oh = (tok[:, t:t + 1] == iota).astype(jnp.float32)
        gates = (lax.dot_general(oh, pf, (((1,), (0,)), ((), ())))
                 + lax.dot_general(h, whhf, (((1,), (1,)), ((), ()))))
        i_ = jax.nn.sigmoid(gates[:, 0:16])
        f_ = jax.nn.sigmoid(gates[:, 16:32])
        g_ = jnp.tanh(gates[:, 32:48])
        o_ = jax.nn.sigmoid(gates[:, 48:64])
        c = f_ * c + i_ * g_
        h = o_ * jnp.tanh(c)
    ohl = (tok[:, L - 1:L] == iota).astype(jnp.float32)
    gb = lax.dot_general(ohl, pb, (((1,), (0,)), ((), ())))
    cb = jax.nn.sigmoid(gb[:, 0:16]) * jnp.tanh(gb[:, 32:48])
    hb = jax.nn.sigmoid(gb[:, 48:64]) * jnp.tanh(cb)
    outact_ref[...] = jnp.concatenate([h, hb], axis=1)
    w1 = w1_ref[...]
    c1 = -jnp.sum(w1[16:32, :], axis=0, keepdims=True)
    xw1_ref[...] = lax.dot_general(h, w1[0:16, :], (((1,), (0,)), ((), ()))) + c1


def _prep_body(xw1_ref, dga_ref, dgb_ref, y1_ref, dinv8_ref):
    deg = 1.0 + dga_ref[...] + dgb_ref[...]
    dinv = lax.rsqrt(deg)
    dinv8_ref[...] = dinv
    d16 = jnp.concatenate([dinv, dinv], axis=1)
    y1_ref[...] = xw1_ref[...] * d16


def _mid_body(t1a_ref, t1b_ref, y1_ref, dinv8_ref, w2_ref, b1_ref, y2a_ref,
              y2b_ref):
    dinv = dinv8_ref[...]
    d16 = jnp.concatenate([dinv, dinv], axis=1)
    y1 = y1_ref[...]
    x1 = _lrelu(d16 * (t1a_ref[...] + t1b_ref[...] + y1) + b1_ref[...])
    xw2 = lax.dot_general(x1, w2_ref[...], (((1,), (0,)), ((), ())))
    y2a_ref[...] = xw2[:, 0:16] * d16
    y2b_ref[...] = xw2[:, 16:32] * d16


def _fin_body(ta0_ref, ta1_ref, tb0_ref, tb1_ref, y2a_ref, y2b_ref, dinv8_ref,
              wfc_ref, b2_ref, bfc_ref, out_ref):
    dinv = dinv8_ref[...]
    d16 = jnp.concatenate([dinv, dinv], axis=1)
    ha = ta0_ref[...] + ta1_ref[...] + y2a_ref[...]
    hb = tb0_ref[...] + tb1_ref[...] + y2b_ref[...]
    t2 = jnp.concatenate([ha * d16, hb * d16], axis=1)
    x2 = _lrelu(t2 + b2_ref[...])
    out_ref[...] = lax.dot_general(x2, wfc_ref[...], (((1,), (0,)), ((), ()))) \
        + bfc_ref[...]


def _blk(shape, imap):
    return pl.BlockSpec(shape, imap)


def _full(shape):
    return pl.BlockSpec(shape, lambda i: tuple(0 for _ in shape))


def kernel(node_sequences, edge_index, emb, W_ih_f, W_hh_f, b_ih_f, b_hh_f,
           W_ih_b, W_hh_b, b_ih_b, b_hh_b, W1, b1, W2, b2, Wfc, bfc):
    f32 = jnp.float32
    src = edge_index[0]
    dst = edge_index[1]
    npad = EPAD - E
    padidx = (jnp.arange(npad, dtype=jnp.int32) % 64) + N
    src2d = jnp.concatenate([src, padidx]).reshape(EC, 128)
    dst2d = jnp.concatenate([dst, padidx]).reshape(EC, 128)
    ones8 = jnp.ones((128, 8), f32)
    zeros8 = jnp.zeros((NP, 8), f32)
    zeros16 = jnp.zeros((NP, 16), f32)

    degp = _deg_kernel(dst2d, ones8, zeros8)

    bf = (b_ih_f + b_hh_f)[None, :]
    bb = (b_ih_b + b_hh_b)[None, :]
    outact, xw1 = pl.pallas_call(
        _enc_body,
        grid=(G,),
        in_specs=[
            _blk((B, L), lambda i: (i, 0)),
            _full((VOCABP, 16)),
            _full((64, 16)),
            _full((64, 16)),
            _full((1, 64)),
            _full((64, 16)),
            _full((1, 64)),
            _full((32, 16)),
        ],
        out_specs=[
            _blk((B, 32), lambda i: (i, 0)),
            _blk((B, 16), lambda i: (i, 0)),
        ],
        out_shape=[
            jax.ShapeDtypeStruct((N, 32), f32),
            jax.ShapeDtypeStruct((NP, 16), f32),
        ],
    )(node_sequences, emb, W_ih_f, W_hh_f, bf, W_ih_b, bb, W1)

    y1, dinv8 = pl.pallas_call(
        _prep_body,
        grid=(G,),
        in_specs=[
            _blk((B, 16), lambda i: (i, 0)),
            _blk((B, 8), lambda i: (i, 0)),
            _blk((B, 8), lambda i: (G + i, 0)),
        ],
        out_specs=[
            _blk((B, 16), lambda i: (i, 0)),
            _blk((B, 8), lambda i: (i, 0)),
        ],
        out_shape=[
            jax.ShapeDtypeStruct((NP, 16), f32),
            jax.ShapeDtypeStruct((NP, 8), f32),
        ],
    )(xw1, degp, degp)

    t1p = _agg_kernel(src2d, dst2d, y1, zeros16, 16)

    y2a, y2b = pl.pallas_call(
        _mid_body,
        grid=(G,),
        in_specs=[
            _blk((B, 16), lambda i: (i, 0)),
            _blk((B, 16), lambda i: (G + i, 0)),
            _blk((B, 16), lambda i: (i, 0)),
            _blk((B, 8), lambda i: (i, 0)),
            _full((16, 32)),
            _full((1, 16)),
        ],
        out_specs=[
            _blk((B, 16), lambda i: (i, 0)),
            _blk((B, 16), lambda i: (i, 0)),
        ],
        out_shape=[
            jax.ShapeDtypeStruct((NP, 16), f32),
            jax.ShapeDtypeStruct((NP, 16), f32),
        ],
    )(t1p, t1p, y1, dinv8, W2, b1[None, :])

    t2pa = _agg_kernel(src2d, dst2d, y2a, zeros16, 16)
    t2pb = _agg_kernel(src2d, dst2d, y2b, zeros16, 16)

    out = pl.pallas_call(
        _fin_body,
        grid=(G,),
        in_specs=[
            _blk((B, 16), lambda i: (i, 0)),
            _blk((B, 16), lambda i: (G + i, 0)),
            _blk((B, 16), lambda i: (i, 0)),
            _blk((B, 16), lambda i: (G + i, 0)),
            _blk((B, 16), lambda i: (i, 0)),
            _blk((B, 16), lambda i: (i, 0)),
            _blk((B, 8), lambda i: (i, 0)),
            _full((32, 20)),
            _full((1, 32)),
            _full((1, 20)),
        ],
        out_specs=_blk((B, 20), lambda i: (i, 0)),
        out_shape=jax.ShapeDtypeStruct((N, 20), f32),
    )(t2pa, t2pa, t2pb, t2pb, y2a, y2b, dinv8, Wfc, b2[None, :], bfc[None, :])

    return (out, outact)
